# baseline (speedup 1.0000x reference)
import jax
import jax.numpy as jnp
from jax.experimental import pallas as pl
from jax.experimental.pallas import tpu as pltpu


def _pick_block(n: int, target: int) -> int:
    best = None
    for b in range(8, min(n, target) + 1, 8):
        if n % b == 0:
            best = b
    return best if best is not None else n


_N_STREAMS = 2
_ROWS_PER_STREAM = 200


def _gcn_body(*refs):
    k = len(refs) - 4
    adj_refs = refs[:k]
    x_ref, w_ref, b_ref, out_ref = refs[k:]
    s = adj_refs[0].shape[0]
    for j, a_ref in enumerate(adj_refs):
        g = jnp.dot(a_ref[...], x_ref[...], preferred_element_type=jnp.float32)
        out_ref[j * s : (j + 1) * s, :] = (
            jnp.dot(g, w_ref[...], preferred_element_type=jnp.float32)
            + b_ref[...]
        )


@jax.jit
def kernel(x, adj_mat, weight, bias):
    n, d_in = x.shape
    d_out = weight.shape[1]
    k, s = _N_STREAMS, _ROWS_PER_STREAM
    if n % (k * s) != 0:
        k, s = 1, _pick_block(n, 400)
    bias2 = bias.reshape(1, d_out)
    adj_specs = [
        pl.BlockSpec((s, n), lambda i, j=j: (k * i + j, 0)) for j in range(k)
    ]
    out = pl.pallas_call(
        _gcn_body,
        grid=(n // (k * s),),
        in_specs=adj_specs
        + [
            pl.BlockSpec((n, d_in), lambda i: (0, 0)),
            pl.BlockSpec((d_in, d_out), lambda i: (0, 0)),
            pl.BlockSpec((1, d_out), lambda i: (0, 0)),
        ],
        out_specs=pl.BlockSpec((k * s, d_out), lambda i: (i, 0)),
        out_shape=jax.ShapeDtypeStruct((n, d_out), jnp.float32),
        compiler_params=pltpu.CompilerParams(
            dimension_semantics=("arbitrary",),
            vmem_limit_bytes=128 * 1024 * 1024,
        ),
    )(*([adj_mat] * k), x, weight, bias2)
    return out
